# baseline (speedup 1.0000x reference)
import jax
import jax.numpy as jnp
from jax import lax
from jax.experimental import pallas as pl
from jax.experimental.pallas import tpu as pltpu
from jax.experimental.pallas import tpu_sc as plsc

D = 128
B = 16384
L = 16
NW = 32
BPW = B // NW
NCHUNK = 4
CHUNK = BPW // NCHUNK
NSEG = D // L
WB = 256
NWB = BPW // WB
GRP = WB // L


def _sc_kernel(mem_hbm, idx_hbm, td_hbm, w_hbm, b_hbm, out_hbm,
               idx_v, td_v, w_v, b_v, rows_v, sem_g, sem_w, sem_s):
    wid = lax.axis_index("s") * 2 + lax.axis_index("c")
    base = wid * BPW

    pltpu.sync_copy(idx_hbm.at[pl.ds(wid * NCHUNK, NCHUNK)], idx_v)

    bounds = [0, 128, 256, 384, 512]
    gathers = []
    for lo, hi in zip(bounds[:-1], bounds[1:]):
        gathers.append(pltpu.make_async_copy(
            mem_hbm.at[idx_v.at[lo // CHUNK, pl.ds(lo % CHUNK, hi - lo)]],
            rows_v.at[pl.ds(lo, hi - lo)],
            sem_g,
        ))
    for c in gathers:
        c.start()

    td_cp = pltpu.make_async_copy(td_hbm.at[pl.ds(base, BPW)], td_v, sem_s)
    w_cp = pltpu.make_async_copy(w_hbm, w_v, sem_s)
    b_cp = pltpu.make_async_copy(b_hbm, b_v, sem_s)
    td_cp.start()
    w_cp.start()
    b_cp.start()
    td_cp.wait()
    w_cp.wait()
    b_cp.wait()

    wsegs = [w_v[pl.ds(j * L, L)] for j in range(NSEG)]
    bsegs = [1.0 + b_v[pl.ds(j * L, L)] for j in range(NSEG)]
    lane_splats = [jnp.full((L,), r, jnp.int32) for r in range(L)]

    writebacks = [
        pltpu.make_async_copy(
            rows_v.at[pl.ds(p * WB, WB)],
            out_hbm.at[pl.ds(base + p * WB, WB)],
            sem_w,
        )
        for p in range(NWB)
    ]

    for j, (lo, hi) in enumerate(zip(bounds[:-1], bounds[1:])):
        gathers[j].wait()
        for p in range(lo // WB, hi // WB):

            def group_body(g, carry, _p=p):
                row0 = _p * WB + g * L
                t_vec = td_v[pl.ds(row0, L)]
                for r in range(L):
                    t16 = t_vec.at[lane_splats[r]].get(mode="promise_in_bounds")
                    i = row0 + r
                    for s in range(NSEG):
                        seg = rows_v[i, pl.ds(s * L, L)]
                        rows_v[i, pl.ds(s * L, L)] = seg * (t16 * wsegs[s] + bsegs[s])
                return carry

            lax.fori_loop(0, WB // L, group_body, 0)
            writebacks[p].start()

    for c in writebacks:
        c.wait()


@jax.jit
def _run(memory, idx2d, td, w, b):
    mesh = plsc.VectorSubcoreMesh(core_axis_name="c", subcore_axis_name="s")
    return pl.kernel(
        _sc_kernel,
        out_type=jax.ShapeDtypeStruct((B, D), jnp.float32),
        mesh=mesh,
        scratch_types=[
            pltpu.VMEM((NCHUNK, CHUNK), jnp.int32),
            pltpu.VMEM((BPW,), jnp.float32),
            pltpu.VMEM((D,), jnp.float32),
            pltpu.VMEM((D,), jnp.float32),
            pltpu.VMEM((BPW, D), jnp.float32),
            pltpu.SemaphoreType.DMA,
            pltpu.SemaphoreType.DMA,
            pltpu.SemaphoreType.DMA,
        ],
    )(memory, idx2d, td, w, b)


def kernel(memory, source_nodes, timestamps, n_layers, time_diffs, W, b):
    idx2d = source_nodes.astype(jnp.int32).reshape(NW * NCHUNK, CHUNK)
    w = W.reshape(D)
    return _run(memory, idx2d, time_diffs, w, b)


# speedup vs baseline: 1.0207x; 1.0207x over previous
import jax
import jax.numpy as jnp
from jax import lax
from jax.experimental import pallas as pl
from jax.experimental.pallas import tpu as pltpu
from jax.experimental.pallas import tpu_sc as plsc

D = 128
B = 16384
L = 16
NW = 32
BPW = B // NW
NCHUNK = 4
CHUNK = BPW // NCHUNK
NSEG = D // L
WB = 256
NWB = BPW // WB
GRP = WB // L


def _sc_kernel(mem_hbm, idx_hbm, td_hbm, w_hbm, b_hbm, out_hbm,
               idx_v, td_v, w_v, b_v, rows_v, sem_g, sem_w, sem_s):
    wid = lax.axis_index("s") * 2 + lax.axis_index("c")
    base = wid * BPW

    td_cp = pltpu.make_async_copy(td_hbm.at[pl.ds(base, BPW)], td_v, sem_s)
    w_cp = pltpu.make_async_copy(w_hbm, w_v, sem_s)
    b_cp = pltpu.make_async_copy(b_hbm, b_v, sem_s)
    td_cp.start()
    w_cp.start()
    b_cp.start()
    pltpu.sync_copy(idx_hbm.at[pl.ds(wid * NCHUNK, NCHUNK)], idx_v)

    bounds = [0, 128, 256, 384, 512]
    gathers = []
    for lo, hi in zip(bounds[:-1], bounds[1:]):
        gathers.append(pltpu.make_async_copy(
            mem_hbm.at[idx_v.at[lo // CHUNK, pl.ds(lo % CHUNK, hi - lo)]],
            rows_v.at[pl.ds(lo, hi - lo)],
            sem_g,
        ))
    for c in gathers:
        c.start()

    td_cp.wait()
    w_cp.wait()
    b_cp.wait()

    wsegs = [w_v[pl.ds(j * L, L)] for j in range(NSEG)]
    bsegs = [1.0 + b_v[pl.ds(j * L, L)] for j in range(NSEG)]
    lane_splats = [jnp.full((L,), r, jnp.int32) for r in range(L)]

    writebacks = [
        pltpu.make_async_copy(
            rows_v.at[pl.ds(p * WB, WB)],
            out_hbm.at[pl.ds(base + p * WB, WB)],
            sem_w,
        )
        for p in range(NWB)
    ]

    for j, (lo, hi) in enumerate(zip(bounds[:-1], bounds[1:])):
        gathers[j].wait()
        for p in range(lo // WB, hi // WB):

            def group_body(g, carry, _p=p):
                row0 = _p * WB + g * L
                t_vec = td_v[pl.ds(row0, L)]
                for r in range(L):
                    t16 = t_vec.at[lane_splats[r]].get(mode="promise_in_bounds")
                    i = row0 + r
                    for s in range(NSEG):
                        seg = rows_v[i, pl.ds(s * L, L)]
                        rows_v[i, pl.ds(s * L, L)] = seg * (t16 * wsegs[s] + bsegs[s])
                return carry

            lax.fori_loop(0, WB // L, group_body, 0)
            writebacks[p].start()

    for c in writebacks:
        c.wait()


@jax.jit
def _run(memory, idx2d, td, w, b):
    mesh = plsc.VectorSubcoreMesh(core_axis_name="c", subcore_axis_name="s")
    return pl.kernel(
        _sc_kernel,
        out_type=jax.ShapeDtypeStruct((B, D), jnp.float32),
        mesh=mesh,
        scratch_types=[
            pltpu.VMEM((NCHUNK, CHUNK), jnp.int32),
            pltpu.VMEM((BPW,), jnp.float32),
            pltpu.VMEM((D,), jnp.float32),
            pltpu.VMEM((D,), jnp.float32),
            pltpu.VMEM((BPW, D), jnp.float32),
            pltpu.SemaphoreType.DMA,
            pltpu.SemaphoreType.DMA,
            pltpu.SemaphoreType.DMA,
        ],
    )(memory, idx2d, td, w, b)


def kernel(memory, source_nodes, timestamps, n_layers, time_diffs, W, b):
    idx2d = source_nodes.astype(jnp.int32).reshape(NW * NCHUNK, CHUNK)
    w = W.reshape(D)
    return _run(memory, idx2d, time_diffs, w, b)
